# baseline (speedup 1.0000x reference)
import jax
import jax.numpy as jnp
from jax.experimental import pallas as pl
from jax.experimental.pallas import tpu as pltpu


def _sel(H, A):
    h = jnp.arange(H)[:, None, None, None]
    kh = jnp.arange(5)[None, :, None, None]
    a = jnp.arange(A)[None, None, :, None]
    p = jnp.arange(2)[None, None, None, :]
    return (h == kh + 2 * a + p).astype(jnp.float32)


def _toeplitz(w, spec, W, C2, row_pad, col_pad):
    S = _sel(W, C2)
    T = jnp.einsum('ozkl,wlcq->' + spec, w.astype(jnp.float32), S)
    rows = T.shape[1] * T.shape[2]
    T = T.reshape(5, rows, 2, C2 * w.shape[0])
    T = jnp.pad(T, ((0, 0), (0, row_pad), (0, 0), (0, col_pad)))
    return T.reshape(5 * (rows + row_pad), 256)


def _fused_net_kernel(x_ref, wt1_ref, b1_ref, wt2_ref, b2_ref,
                      wf1_ref, bf1_ref, wf2_ref, bf2_ref, wf3_ref, bf3_ref,
                      o_ref):
    NB = x_ref.shape[1]
    x = x_ref[...]

    a1_parts = []
    for j in range(7):
        z = jnp.dot(x[4 * j:4 * j + 4].reshape(4 * NB, 96), wt1_ref[0],
                    preferred_element_type=jnp.float32)
        for k in range(1, 5):
            z = z + jnp.dot(x[4 * j + k:4 * j + k + 4].reshape(4 * NB, 96),
                            wt1_ref[k], preferred_element_type=jnp.float32)
        zw = jnp.maximum(z[:, :128], z[:, 128:]).reshape(2, 2, NB, 128)
        a = jnp.maximum(zw[:, 0], zw[:, 1])
        a1_parts.append(
            jnp.maximum(a + b1_ref[...], 0.0).astype(jnp.bfloat16))
    a1 = jnp.concatenate(a1_parts, axis=0)

    p2 = jnp.concatenate([a1[k:k + 10] for k in range(5)], axis=2)
    z2 = jnp.dot(p2.reshape(10 * NB, 640), wt2_ref[...],
                 preferred_element_type=jnp.float32)
    z2w = jnp.maximum(z2[:, :128], z2[:, 128:]).reshape(5, 2, NB, 128)
    a2 = jnp.maximum(z2w[:, 0], z2w[:, 1])
    a2 = jnp.maximum(a2 + b2_ref[...], 0.0).astype(jnp.bfloat16)

    h1 = jnp.dot(a2[0], wf1_ref[0], preferred_element_type=jnp.float32)
    for k in range(1, 5):
        h1 = h1 + jnp.dot(a2[k], wf1_ref[k],
                          preferred_element_type=jnp.float32)
    h1 = jnp.maximum(h1 + bf1_ref[...], 0.0).astype(jnp.bfloat16)
    h2 = jnp.dot(h1, wf2_ref[...], preferred_element_type=jnp.float32)
    h2 = jnp.maximum(h2 + bf2_ref[...], 0.0).astype(jnp.bfloat16)
    y = jnp.dot(h2, wf3_ref[...], preferred_element_type=jnp.float32)
    o_ref[...] = y + bf3_ref[...]


def kernel(x, conv1_w, conv1_b, conv2_w, conv2_b,
           fc1_w, fc1_b, fc2_w, fc2_b, fc3_w, fc3_b):
    B = x.shape[0]
    NB = min(512, B)
    grid = B // NB

    xh = x.transpose(2, 0, 1, 3).reshape(32, B, 96).astype(jnp.bfloat16)

    wt1 = _toeplitz(conv1_w, 'kzwqco', 32, 14, 0, 44
                    ).reshape(5, 96, 256).astype(jnp.bfloat16)
    wt2 = _toeplitz(conv2_w, 'kwzqco', 14, 5, 44, 48).astype(jnp.bfloat16)
    b1l = jnp.pad(jnp.tile(conv1_b.astype(jnp.float32), 14), (0, 44)).reshape(1, 128)
    b2l = jnp.pad(jnp.tile(conv2_b.astype(jnp.float32), 5), (0, 48)).reshape(1, 128)

    rows = jnp.arange(5 * 128)
    a2i, rest = rows // 128, rows % 128
    c2i, o2i = rest // 16, rest % 16
    valid = (c2i < 5) & (rest < 80)
    src = jnp.clip(o2i * 25 + a2i * 5 + c2i, 0, 399)
    wf1 = jnp.where(valid[:, None], fc1_w.T.astype(jnp.float32)[src], 0.0
                    ).reshape(5, 128, 120).astype(jnp.bfloat16)
    wf2 = fc2_w.T.astype(jnp.bfloat16)
    wf3 = fc3_w.T.astype(jnp.bfloat16)
    bf1 = fc1_b.astype(jnp.float32).reshape(1, 120)
    bf2 = fc2_b.astype(jnp.float32).reshape(1, 84)
    bf3 = fc3_b.astype(jnp.float32).reshape(1, 10)

    const = lambda i: (0, 0)
    out = pl.pallas_call(
        _fused_net_kernel,
        out_shape=jax.ShapeDtypeStruct((B, 10), jnp.float32),
        grid=(grid,),
        in_specs=[pl.BlockSpec((32, NB, 96), lambda i: (0, i, 0)),
                  pl.BlockSpec((5, 96, 256), lambda i: (0, 0, 0)),
                  pl.BlockSpec((1, 128), const),
                  pl.BlockSpec((640, 256), const),
                  pl.BlockSpec((1, 128), const),
                  pl.BlockSpec((5, 128, 120), lambda i: (0, 0, 0)),
                  pl.BlockSpec((1, 120), const),
                  pl.BlockSpec((120, 84), const),
                  pl.BlockSpec((1, 84), const),
                  pl.BlockSpec((84, 10), const),
                  pl.BlockSpec((1, 10), const)],
        out_specs=pl.BlockSpec((NB, 10), lambda i: (i, 0)),
        compiler_params=pltpu.CompilerParams(
            dimension_semantics=("parallel",)),
    )(xh, wt1, b1l, wt2, b2l, wf1, bf1, wf2, bf2, wf3, bf3)
    return out


# speedup vs baseline: 1.2024x; 1.2024x over previous
import jax
import jax.numpy as jnp
from jax.experimental import pallas as pl
from jax.experimental.pallas import tpu as pltpu


def _sel(H, A):
    h = jnp.arange(H)[:, None, None, None]
    kh = jnp.arange(5)[None, :, None, None]
    a = jnp.arange(A)[None, None, :, None]
    p = jnp.arange(2)[None, None, None, :]
    return (h == kh + 2 * a + p).astype(jnp.float32)


def _toeplitz(w, spec, W, C2, row_pad, col_pad):
    S = _sel(W, C2)
    T = jnp.einsum('ozkl,wlcq->' + spec, w.astype(jnp.float32), S)
    rows = T.shape[1] * T.shape[2]
    T = T.reshape(5, rows, 2, C2 * w.shape[0])
    T = jnp.pad(T, ((0, 0), (0, row_pad), (0, 0), (0, col_pad)))
    return T.reshape(5 * (rows + row_pad), 256)


def _fused_net_kernel(x_ref, wt1_ref, b1_ref, wt2_ref, b2_ref,
                      wf1_ref, bf1_ref, wf2_ref, bf2_ref, wf3_ref, bf3_ref,
                      o_ref):
    NB = x_ref.shape[1]
    x = x_ref[...]

    p1 = jnp.concatenate([x[k:k + 28] for k in range(5)], axis=2)
    z1 = jnp.dot(p1.reshape(28 * NB, 480), wt1_ref[...],
                 preferred_element_type=jnp.float32)
    zw = jnp.maximum(z1[:, :128], z1[:, 128:]).reshape(14, 2, NB, 128)
    a1 = jnp.maximum(zw[:, 0], zw[:, 1])
    a1 = jnp.maximum(a1 + b1_ref[...], 0.0).astype(jnp.bfloat16)

    p2 = jnp.concatenate([a1[k:k + 10] for k in range(5)], axis=2)
    z2 = jnp.dot(p2.reshape(10 * NB, 640), wt2_ref[...],
                 preferred_element_type=jnp.float32)
    z2w = jnp.maximum(z2[:, :128], z2[:, 128:]).reshape(5, 2, NB, 128)
    a2 = jnp.maximum(z2w[:, 0], z2w[:, 1])
    a2 = jnp.maximum(a2 + b2_ref[...], 0.0).astype(jnp.bfloat16)

    h1 = jnp.dot(a2[0], wf1_ref[0], preferred_element_type=jnp.float32)
    for k in range(1, 5):
        h1 = h1 + jnp.dot(a2[k], wf1_ref[k],
                          preferred_element_type=jnp.float32)
    h1 = jnp.maximum(h1 + bf1_ref[...], 0.0).astype(jnp.bfloat16)
    h2 = jnp.dot(h1, wf2_ref[...], preferred_element_type=jnp.float32)
    h2 = jnp.maximum(h2 + bf2_ref[...], 0.0).astype(jnp.bfloat16)
    y = jnp.dot(h2, wf3_ref[...], preferred_element_type=jnp.float32)
    o_ref[...] = y + bf3_ref[...]


def kernel(x, conv1_w, conv1_b, conv2_w, conv2_b,
           fc1_w, fc1_b, fc2_w, fc2_b, fc3_w, fc3_b):
    B = x.shape[0]
    NB = min(512, B)
    grid = B // NB

    xh = x.transpose(2, 0, 1, 3).reshape(32, B, 96).astype(jnp.bfloat16)

    wt1 = _toeplitz(conv1_w, 'kzwqco', 32, 14, 0, 44).astype(jnp.bfloat16)
    wt2 = _toeplitz(conv2_w, 'kwzqco', 14, 5, 44, 48).astype(jnp.bfloat16)
    b1l = jnp.pad(jnp.tile(conv1_b.astype(jnp.float32), 14), (0, 44)).reshape(1, 128)
    b2l = jnp.pad(jnp.tile(conv2_b.astype(jnp.float32), 5), (0, 48)).reshape(1, 128)

    rows = jnp.arange(5 * 128)
    a2i, rest = rows // 128, rows % 128
    c2i, o2i = rest // 16, rest % 16
    valid = (c2i < 5) & (rest < 80)
    src = jnp.clip(o2i * 25 + a2i * 5 + c2i, 0, 399)
    wf1 = jnp.where(valid[:, None], fc1_w.T.astype(jnp.float32)[src], 0.0
                    ).reshape(5, 128, 120).astype(jnp.bfloat16)
    wf2 = fc2_w.T.astype(jnp.bfloat16)
    wf3 = fc3_w.T.astype(jnp.bfloat16)
    bf1 = fc1_b.astype(jnp.float32).reshape(1, 120)
    bf2 = fc2_b.astype(jnp.float32).reshape(1, 84)
    bf3 = fc3_b.astype(jnp.float32).reshape(1, 10)

    const = lambda i: (0, 0)
    out = pl.pallas_call(
        _fused_net_kernel,
        out_shape=jax.ShapeDtypeStruct((B, 10), jnp.float32),
        grid=(grid,),
        in_specs=[pl.BlockSpec((32, NB, 96), lambda i: (0, i, 0)),
                  pl.BlockSpec((480, 256), const),
                  pl.BlockSpec((1, 128), const),
                  pl.BlockSpec((640, 256), const),
                  pl.BlockSpec((1, 128), const),
                  pl.BlockSpec((5, 128, 120), lambda i: (0, 0, 0)),
                  pl.BlockSpec((1, 120), const),
                  pl.BlockSpec((120, 84), const),
                  pl.BlockSpec((1, 84), const),
                  pl.BlockSpec((84, 10), const),
                  pl.BlockSpec((1, 10), const)],
        out_specs=pl.BlockSpec((NB, 10), lambda i: (i, 0)),
        compiler_params=pltpu.CompilerParams(
            dimension_semantics=("parallel",)),
    )(xh, wt1, b1l, wt2, b2l, wf1, bf1, wf2, bf2, wf3, bf3)
    return out
